# baseline (speedup 1.0000x reference)
import functools

import jax
import jax.numpy as jnp
from jax import lax
from jax.experimental import pallas as pl
from jax.experimental.pallas import tpu as pltpu
from jax.experimental.pallas import tpu_sc as plsc

TOPK = 2


def _sc_mesh():
    return plsc.VectorSubcoreMesh(core_axis_name="c", subcore_axis_name="s")


def _dispatch_sc(xf, d0, d1, TS):
    T, D = xf.shape
    SL = 8
    LN = D // SL
    x3 = xf.reshape(T, SL, LN)

    NW = 32
    C = 32
    K = T // (NW * C)

    @functools.partial(
        pl.kernel,
        out_type=jax.ShapeDtypeStruct((TS, SL, LN), xf.dtype),
        scratch_types=[
            pltpu.VMEM((C, SL, LN), xf.dtype),
            pltpu.VMEM((C, SL, LN), xf.dtype),
            pltpu.VMEM((C,), jnp.int32),
            pltpu.VMEM((C,), jnp.int32),
            pltpu.VMEM((C,), jnp.int32),
            pltpu.VMEM((C,), jnp.int32),
            pltpu.SemaphoreType.DMA,
            pltpu.SemaphoreType.DMA,
            pltpu.SemaphoreType.DMA,
            pltpu.SemaphoreType.DMA,
        ],
        mesh=_sc_mesh(),
    )
    def disp(x_hbm, d0_hbm, d1_hbm, o_hbm, xa_v, xb_v, i0a, i0b, i1a, i1b,
             l0, l1, s0, s1):
        wid = lax.axis_index("s") * 2 + lax.axis_index("c")
        base = wid * (K * C)
        xbufs = (xa_v, xb_v)
        i0s = (i0a, i0b)
        i1s = (i1a, i1b)
        lsems = (l0, l1)
        ssems = (s0, s1)
        loads = {}
        scats = {}
        for j in range(K):
            b = j % 2
            st = base + j * C
            if j >= 2:
                sa, sb = scats[j - 2]
                sa.wait()
                sb.wait()
            loads[j] = pltpu.async_copy(x_hbm.at[pl.ds(st, C)], xbufs[b],
                                        lsems[b])
            pltpu.sync_copy(d0_hbm.at[pl.ds(st, C)], i0s[b])
            pltpu.sync_copy(d1_hbm.at[pl.ds(st, C)], i1s[b])
            if j >= 1:
                p = (j - 1) % 2
                loads[j - 1].wait()
                scats[j - 1] = (
                    pltpu.async_copy(xbufs[p], o_hbm.at[i0s[p]], ssems[p]),
                    pltpu.async_copy(xbufs[p], o_hbm.at[i1s[p]], ssems[p]),
                )
        loads[K - 1].wait()
        if K >= 2:
            sa, sb = scats[K - 2]
            sa.wait()
            sb.wait()
        p = (K - 1) % 2
        fa = pltpu.async_copy(xbufs[p], o_hbm.at[i0s[p]], ssems[p])
        fb = pltpu.async_copy(xbufs[p], o_hbm.at[i1s[p]], ssems[p])
        fa.wait()
        fb.wait()

    return disp(x3, d0, d1).reshape(TS, D)


def _gather_sc(Ys, idx):
    TS, D = Ys.shape
    SL = 8
    LN = D // SL
    N = idx.shape[0]
    y3 = Ys.reshape(TS, SL, LN)

    NW = 32
    C = 32
    K = N // (NW * C)

    @functools.partial(
        pl.kernel,
        out_type=jax.ShapeDtypeStruct((N, SL, LN), jnp.float32),
        scratch_types=[
            pltpu.VMEM((K * C,), jnp.int32),
            pltpu.VMEM((C, SL, LN), jnp.float32),
            pltpu.VMEM((C, SL, LN), jnp.float32),
            pltpu.SemaphoreType.DMA,
            pltpu.SemaphoreType.DMA,
            pltpu.SemaphoreType.DMA,
            pltpu.SemaphoreType.DMA,
        ],
        mesh=_sc_mesh(),
    )
    def gath(y_hbm, i_hbm, o_hbm, i_v, r0_v, r1_v, g0, g1, o0, o1):
        wid = lax.axis_index("s") * 2 + lax.axis_index("c")
        base = wid * (K * C)
        pltpu.sync_copy(i_hbm.at[pl.ds(base, K * C)], i_v)
        bufs = (r0_v, r1_v)
        gsems = (g0, g1)
        osems = (o0, o1)
        gets = {}
        puts = {}
        for j in range(K):
            b = j % 2
            if j >= 2:
                puts[j - 2].wait()
            gets[j] = pltpu.async_copy(
                y_hbm.at[i_v.at[pl.ds(j * C, C)]], bufs[b], gsems[b])
            if j >= 1:
                gets[j - 1].wait()
                puts[j - 1] = pltpu.async_copy(
                    bufs[(j - 1) % 2],
                    o_hbm.at[pl.ds(base + (j - 1) * C, C)],
                    osems[(j - 1) % 2])
        gets[K - 1].wait()
        if K >= 2:
            puts[K - 2].wait()
        pltpu.async_copy(
            bufs[(K - 1) % 2], o_hbm.at[pl.ds(base + (K - 1) * C, C)],
            osems[(K - 1) % 2]).wait()

    return gath(y3, idx).reshape(N, D)


def _gating(logits_in):
    T, E = logits_in.shape
    BK = min(512, T)
    GT = T // BK

    def body(l_ref, tw_ref, ti_ref, c_ref, acc_ref):
        t = pl.program_id(0)

        @pl.when(t == 0)
        def _():
            acc_ref[...] = jnp.zeros_like(acc_ref)

        logits = l_ref[...]
        iota_e = lax.broadcasted_iota(jnp.int32, (BK, E), 1)
        m1 = jnp.max(logits, axis=-1, keepdims=True)
        i1 = jnp.min(jnp.where(logits == m1, iota_e, E), axis=-1, keepdims=True)
        l2 = jnp.where(iota_e == i1, -jnp.inf, logits)
        m2 = jnp.max(l2, axis=-1, keepdims=True)
        i2 = jnp.min(jnp.where(l2 == m2, iota_e, E), axis=-1, keepdims=True)
        e2 = jnp.exp(m2 - m1)
        w0 = 1.0 / (1.0 + e2)
        w1 = e2 / (1.0 + e2)
        tw_ref[...] = jnp.concatenate([w0, w1], axis=1)
        ti_ref[...] = jnp.concatenate([i1, i2], axis=1).astype(jnp.int32)
        onehot = ((iota_e == i1) | (iota_e == i2)).astype(jnp.float32)
        acc_ref[...] += jnp.sum(onehot, axis=0, keepdims=True)

        @pl.when(t == GT - 1)
        def _():
            c_ref[...] = acc_ref[...]

    return pl.pallas_call(
        body,
        grid=(GT,),
        in_specs=[
            pl.BlockSpec((BK, E), lambda t: (t, 0)),
        ],
        out_specs=[
            pl.BlockSpec((BK, TOPK), lambda t: (t, 0)),
            pl.BlockSpec((BK, TOPK), lambda t: (t, 0)),
            pl.BlockSpec((1, E), lambda t: (0, 0)),
        ],
        out_shape=[
            jax.ShapeDtypeStruct((T, TOPK), jnp.float32),
            jax.ShapeDtypeStruct((T, TOPK), jnp.int32),
            jax.ShapeDtypeStruct((1, E), jnp.float32),
        ],
        scratch_shapes=[pltpu.VMEM((1, E), jnp.float32)],
    )(logits_in)


def _routing(flat_e, off_f, E):
    TS = flat_e.shape[0]
    BK = min(1024, TS)
    GT = TS // BK

    def body(e_ref, off_ref, dest_ref, carry_ref):
        t = pl.program_id(0)

        @pl.when(t == 0)
        def _():
            carry_ref[...] = jnp.zeros_like(carry_ref)

        iota_e = lax.broadcasted_iota(jnp.int32, (BK, E), 1)
        onehot = (e_ref[...] == iota_e).astype(jnp.float32)
        li = lax.broadcasted_iota(jnp.int32, (BK, BK), 0)
        lj = lax.broadcasted_iota(jnp.int32, (BK, BK), 1)
        lexc = (lj < li).astype(jnp.float32)
        intra = lax.dot_general(
            lexc, onehot, (((1,), (0,)), ((), ())),
            preferred_element_type=jnp.float32,
            precision=lax.Precision.HIGHEST,
        )
        pos = carry_ref[...] + intra + off_ref[...]
        dest = jnp.sum(onehot * pos, axis=-1, keepdims=True)
        dest_ref[...] = dest.astype(jnp.int32)
        carry_ref[...] += jnp.sum(onehot, axis=0, keepdims=True)

    return pl.pallas_call(
        body,
        grid=(GT,),
        in_specs=[
            pl.BlockSpec((BK, 1), lambda t: (t, 0)),
            pl.BlockSpec((1, E), lambda t: (0, 0)),
        ],
        out_specs=pl.BlockSpec((BK, 1), lambda t: (t, 0)),
        out_shape=jax.ShapeDtypeStruct((TS, 1), jnp.int32),
        scratch_shapes=[pltpu.VMEM((1, E), jnp.float32)],
    )(flat_e, off_f)


def _ffn(Xs, W1, W2, W3, tabs, M, NB, NE):
    TS, D = Xs.shape
    E, H, _ = W1.shape
    eb, ee, es, en, ef = tabs

    def body(eb_r, ee_r, es_r, en_r, ef_r, x_ref, w1_ref, w3_ref, w2_ref,
             y_ref):
        j = pl.program_id(0)
        row = eb_r[j] * M + lax.broadcasted_iota(jnp.int32, (M, 1), 0)
        valid = (row >= es_r[j]) & (row < en_r[j])
        xm = jnp.where(valid, x_ref[...], 0.0)
        z1 = lax.dot_general(xm, w1_ref[0], (((1,), (1,)), ((), ())),
                             preferred_element_type=jnp.float32)
        z3 = lax.dot_general(xm, w3_ref[0], (((1,), (1,)), ((), ())),
                             preferred_element_type=jnp.float32)
        hh = (z1 * z3) / (1.0 + jnp.exp(-z1))
        y = lax.dot_general(hh, w2_ref[0], (((1,), (1,)), ((), ())),
                            preferred_element_type=jnp.float32)

        @pl.when(ef_r[j] == 1)
        def _():
            y_ref[...] = y

        @pl.when(ef_r[j] == 0)
        def _():
            y_ref[...] += y

    grid_spec = pltpu.PrefetchScalarGridSpec(
        num_scalar_prefetch=5,
        grid=(NE,),
        in_specs=[
            pl.BlockSpec((M, D), lambda j, eb, ee, es, en, ef: (eb[j], 0)),
            pl.BlockSpec((1, H, D), lambda j, eb, ee, es, en, ef: (ee[j], 0, 0),
                         pipeline_mode=pl.Buffered(buffer_count=1)),
            pl.BlockSpec((1, H, D), lambda j, eb, ee, es, en, ef: (ee[j], 0, 0),
                         pipeline_mode=pl.Buffered(buffer_count=1)),
            pl.BlockSpec((1, D, H), lambda j, eb, ee, es, en, ef: (ee[j], 0, 0),
                         pipeline_mode=pl.Buffered(buffer_count=2)),
        ],
        out_specs=pl.BlockSpec((M, D), lambda j, eb, ee, es, en, ef: (eb[j], 0)),
    )
    return pl.pallas_call(
        body,
        grid_spec=grid_spec,
        out_shape=jax.ShapeDtypeStruct((TS, D), jnp.float32),
        compiler_params=pltpu.CompilerParams(
            dimension_semantics=("arbitrary",),
        ),
    )(eb, ee, es, en, ef, Xs, W1, W3, W2)


def _weighted_add(G3, tw):
    T, _, D = G3.shape
    BK = min(512, T)

    def body(g_ref, w_ref, o_ref):
        g = g_ref[...]
        w = w_ref[...]
        o_ref[...] = g[:, 0, :] * w[:, 0:1] + g[:, 1, :] * w[:, 1:2]

    return pl.pallas_call(
        body,
        grid=(T // BK,),
        in_specs=[
            pl.BlockSpec((BK, TOPK, D), lambda t: (t, 0, 0)),
            pl.BlockSpec((BK, TOPK), lambda t: (t, 0)),
        ],
        out_specs=pl.BlockSpec((BK, D), lambda t: (t, 0)),
        out_shape=jax.ShapeDtypeStruct((T, D), jnp.float32),
    )(G3, tw)


def _entry_tables(off_i, M, NB, NE, E):
    cnt = off_i[1:] - off_i[:-1]
    fb = jnp.where(cnt > 0, off_i[:-1] // M, 0)
    lb = jnp.where(cnt > 0, (off_i[1:] - 1) // M, -1)
    ne = jnp.maximum(lb - fb + 1, 0)
    basec = jnp.concatenate([jnp.zeros((1,), jnp.int32), jnp.cumsum(ne)])
    total = basec[E]
    j = jnp.arange(NE, dtype=jnp.int32)
    e_j = jnp.sum((j[:, None] >= basec[None, 1:]).astype(jnp.int32), axis=1)
    valid = j < total
    e_c = jnp.minimum(e_j, E - 1)
    blk = fb[e_c] + (j - basec[e_c])
    blk = jnp.where(valid, blk, NB - 1)
    es = jnp.where(valid, off_i[e_c], 0)
    en = jnp.where(valid, off_i[e_c + 1], 0)
    ee = jnp.where(valid, e_c, 0)
    prev = jnp.concatenate([jnp.full((1,), -1, jnp.int32), blk[:-1]])
    ef = ((blk != prev) & valid).astype(jnp.int32)
    return (blk.astype(jnp.int32), ee.astype(jnp.int32), es.astype(jnp.int32),
            en.astype(jnp.int32), ef)


def kernel(x, gate_w, W1, W2, W3):
    B, S, D = x.shape
    E = gate_w.shape[0]
    T = B * S
    TS = T * TOPK
    M = min(256, TS)
    NB = TS // M
    NE = NB + E - 1

    xf = x.reshape(T, D)
    logits = xf @ gate_w.T
    tw, ti, counts = _gating(logits)

    counts_i = counts[0].astype(jnp.int32)
    off_i = jnp.concatenate(
        [jnp.zeros((1,), jnp.int32), jnp.cumsum(counts_i)])
    off_f = off_i[:-1].astype(jnp.float32).reshape(1, E)

    flat_e = ti.reshape(TS, 1)
    dest = _routing(flat_e, off_f, E)
    dest_flat = dest.reshape(TS)

    d2 = dest.reshape(T, TOPK)
    Xs = _dispatch_sc(xf, d2[:, 0], d2[:, 1], TS)

    tabs = _entry_tables(off_i, M, NB, NE, E)
    Ys = _ffn(Xs, W1, W2, W3, tabs, M, NB, NE)

    G = _gather_sc(Ys, dest_flat)
    y = _weighted_add(G.reshape(T, TOPK, D), tw)
    return y.reshape(B, S, D)


# speedup vs baseline: 1.0211x; 1.0211x over previous
import functools

import jax
import jax.numpy as jnp
from jax import lax
from jax.experimental import pallas as pl
from jax.experimental.pallas import tpu as pltpu
from jax.experimental.pallas import tpu_sc as plsc

TOPK = 2


def _sc_mesh():
    return plsc.VectorSubcoreMesh(core_axis_name="c", subcore_axis_name="s")


def _dispatch_sc(xf, d0, d1, TS):
    T, D = xf.shape
    SL = 8
    LN = D // SL
    x3 = xf.reshape(T, SL, LN)

    NW = 32
    C = 32
    K = T // (NW * C)

    @functools.partial(
        pl.kernel,
        out_type=jax.ShapeDtypeStruct((TS, SL, LN), xf.dtype),
        scratch_types=[
            pltpu.VMEM((C, SL, LN), xf.dtype),
            pltpu.VMEM((C, SL, LN), xf.dtype),
            pltpu.VMEM((C,), jnp.int32),
            pltpu.VMEM((C,), jnp.int32),
            pltpu.VMEM((C,), jnp.int32),
            pltpu.VMEM((C,), jnp.int32),
            pltpu.SemaphoreType.DMA,
            pltpu.SemaphoreType.DMA,
            pltpu.SemaphoreType.DMA,
            pltpu.SemaphoreType.DMA,
        ],
        mesh=_sc_mesh(),
    )
    def disp(x_hbm, d0_hbm, d1_hbm, o_hbm, xa_v, xb_v, i0a, i0b, i1a, i1b,
             l0, l1, s0, s1):
        wid = lax.axis_index("s") * 2 + lax.axis_index("c")
        base = wid * (K * C)
        xbufs = (xa_v, xb_v)
        i0s = (i0a, i0b)
        i1s = (i1a, i1b)
        lsems = (l0, l1)
        ssems = (s0, s1)
        loads = {}
        scats = {}
        for j in range(K):
            b = j % 2
            st = base + j * C
            if j >= 2:
                sa, sb = scats[j - 2]
                sa.wait()
                sb.wait()
            loads[j] = pltpu.async_copy(x_hbm.at[pl.ds(st, C)], xbufs[b],
                                        lsems[b])
            pltpu.sync_copy(d0_hbm.at[pl.ds(st, C)], i0s[b])
            pltpu.sync_copy(d1_hbm.at[pl.ds(st, C)], i1s[b])
            if j >= 1:
                p = (j - 1) % 2
                loads[j - 1].wait()
                scats[j - 1] = (
                    pltpu.async_copy(xbufs[p], o_hbm.at[i0s[p]], ssems[p]),
                    pltpu.async_copy(xbufs[p], o_hbm.at[i1s[p]], ssems[p]),
                )
        loads[K - 1].wait()
        if K >= 2:
            sa, sb = scats[K - 2]
            sa.wait()
            sb.wait()
        p = (K - 1) % 2
        fa = pltpu.async_copy(xbufs[p], o_hbm.at[i0s[p]], ssems[p])
        fb = pltpu.async_copy(xbufs[p], o_hbm.at[i1s[p]], ssems[p])
        fa.wait()
        fb.wait()

    return disp(x3, d0, d1).reshape(TS, D)


def _gather_sc(Ys, idx):
    TS, D = Ys.shape
    SL = 8
    LN = D // SL
    N = idx.shape[0]
    y3 = Ys.reshape(TS, SL, LN)

    NW = 32
    C = 32
    K = N // (NW * C)

    @functools.partial(
        pl.kernel,
        out_type=jax.ShapeDtypeStruct((N, SL, LN), jnp.float32),
        scratch_types=[
            pltpu.VMEM((K * C,), jnp.int32),
            pltpu.VMEM((C, SL, LN), jnp.float32),
            pltpu.VMEM((C, SL, LN), jnp.float32),
            pltpu.SemaphoreType.DMA,
            pltpu.SemaphoreType.DMA,
            pltpu.SemaphoreType.DMA,
            pltpu.SemaphoreType.DMA,
        ],
        mesh=_sc_mesh(),
    )
    def gath(y_hbm, i_hbm, o_hbm, i_v, r0_v, r1_v, g0, g1, o0, o1):
        wid = lax.axis_index("s") * 2 + lax.axis_index("c")
        base = wid * (K * C)
        pltpu.sync_copy(i_hbm.at[pl.ds(base, K * C)], i_v)
        bufs = (r0_v, r1_v)
        gsems = (g0, g1)
        osems = (o0, o1)
        gets = {}
        puts = {}
        for j in range(K):
            b = j % 2
            if j >= 2:
                puts[j - 2].wait()
            gets[j] = pltpu.async_copy(
                y_hbm.at[i_v.at[pl.ds(j * C, C)]], bufs[b], gsems[b])
            if j >= 1:
                gets[j - 1].wait()
                puts[j - 1] = pltpu.async_copy(
                    bufs[(j - 1) % 2],
                    o_hbm.at[pl.ds(base + (j - 1) * C, C)],
                    osems[(j - 1) % 2])
        gets[K - 1].wait()
        if K >= 2:
            puts[K - 2].wait()
        pltpu.async_copy(
            bufs[(K - 1) % 2], o_hbm.at[pl.ds(base + (K - 1) * C, C)],
            osems[(K - 1) % 2]).wait()

    return gath(y3, idx).reshape(N, D)


def _gating(logits_in):
    T, E = logits_in.shape
    BK = min(512, T)
    GT = T // BK

    def body(l_ref, tw_ref, ti_ref, c_ref, acc_ref):
        t = pl.program_id(0)

        @pl.when(t == 0)
        def _():
            acc_ref[...] = jnp.zeros_like(acc_ref)

        logits = l_ref[...]
        iota_e = lax.broadcasted_iota(jnp.int32, (BK, E), 1)
        m1 = jnp.max(logits, axis=-1, keepdims=True)
        i1 = jnp.min(jnp.where(logits == m1, iota_e, E), axis=-1, keepdims=True)
        l2 = jnp.where(iota_e == i1, -jnp.inf, logits)
        m2 = jnp.max(l2, axis=-1, keepdims=True)
        i2 = jnp.min(jnp.where(l2 == m2, iota_e, E), axis=-1, keepdims=True)
        e2 = jnp.exp(m2 - m1)
        w0 = 1.0 / (1.0 + e2)
        w1 = e2 / (1.0 + e2)
        tw_ref[...] = jnp.concatenate([w0, w1], axis=1)
        ti_ref[...] = jnp.concatenate([i1, i2], axis=1).astype(jnp.int32)
        onehot = ((iota_e == i1) | (iota_e == i2)).astype(jnp.float32)
        acc_ref[...] += jnp.sum(onehot, axis=0, keepdims=True)

        @pl.when(t == GT - 1)
        def _():
            c_ref[...] = acc_ref[...]

    return pl.pallas_call(
        body,
        grid=(GT,),
        in_specs=[
            pl.BlockSpec((BK, E), lambda t: (t, 0)),
        ],
        out_specs=[
            pl.BlockSpec((BK, TOPK), lambda t: (t, 0)),
            pl.BlockSpec((BK, TOPK), lambda t: (t, 0)),
            pl.BlockSpec((1, E), lambda t: (0, 0)),
        ],
        out_shape=[
            jax.ShapeDtypeStruct((T, TOPK), jnp.float32),
            jax.ShapeDtypeStruct((T, TOPK), jnp.int32),
            jax.ShapeDtypeStruct((1, E), jnp.float32),
        ],
        scratch_shapes=[pltpu.VMEM((1, E), jnp.float32)],
    )(logits_in)


def _routing(flat_e, off_f, E):
    TS = flat_e.shape[0]
    BK = min(512, TS)
    GT = TS // BK

    def body(e_ref, off_ref, dest_ref, carry_ref):
        t = pl.program_id(0)

        @pl.when(t == 0)
        def _():
            carry_ref[...] = jnp.zeros_like(carry_ref)

        iota_e = lax.broadcasted_iota(jnp.int32, (BK, E), 1)
        onehot = (e_ref[...] == iota_e).astype(jnp.float32)
        li = lax.broadcasted_iota(jnp.int32, (BK, BK), 0)
        lj = lax.broadcasted_iota(jnp.int32, (BK, BK), 1)
        lexc = (lj < li).astype(jnp.float32)
        intra = lax.dot_general(
            lexc, onehot, (((1,), (0,)), ((), ())),
            preferred_element_type=jnp.float32,
            precision=lax.Precision.HIGHEST,
        )
        pos = carry_ref[...] + intra + off_ref[...]
        dest = jnp.sum(onehot * pos, axis=-1, keepdims=True)
        dest_ref[...] = dest.astype(jnp.int32)
        carry_ref[...] += jnp.sum(onehot, axis=0, keepdims=True)

    return pl.pallas_call(
        body,
        grid=(GT,),
        in_specs=[
            pl.BlockSpec((BK, 1), lambda t: (t, 0)),
            pl.BlockSpec((1, E), lambda t: (0, 0)),
        ],
        out_specs=pl.BlockSpec((BK, 1), lambda t: (t, 0)),
        out_shape=jax.ShapeDtypeStruct((TS, 1), jnp.int32),
        scratch_shapes=[pltpu.VMEM((1, E), jnp.float32)],
    )(flat_e, off_f)


def _ffn(Xs, W1, W2, W3, tabs, M, NB, NE):
    TS, D = Xs.shape
    E, H, _ = W1.shape
    eb, ee, es, en, ef = tabs

    def body(eb_r, ee_r, es_r, en_r, ef_r, x_ref, w1_ref, w3_ref, w2_ref,
             y_ref):
        j = pl.program_id(0)
        row = eb_r[j] * M + lax.broadcasted_iota(jnp.int32, (M, 1), 0)
        valid = (row >= es_r[j]) & (row < en_r[j])
        xm = jnp.where(valid, x_ref[...], 0.0)
        z1 = lax.dot_general(xm, w1_ref[0], (((1,), (1,)), ((), ())),
                             preferred_element_type=jnp.float32)
        z3 = lax.dot_general(xm, w3_ref[0], (((1,), (1,)), ((), ())),
                             preferred_element_type=jnp.float32)
        hh = z1 * (1.0 / (1.0 + jnp.exp(-z1))) * z3
        y = lax.dot_general(hh, w2_ref[0], (((1,), (1,)), ((), ())),
                            preferred_element_type=jnp.float32)

        @pl.when(ef_r[j] == 1)
        def _():
            y_ref[...] = y

        @pl.when(ef_r[j] == 0)
        def _():
            y_ref[...] += y

    grid_spec = pltpu.PrefetchScalarGridSpec(
        num_scalar_prefetch=5,
        grid=(NE,),
        in_specs=[
            pl.BlockSpec((M, D), lambda j, eb, ee, es, en, ef: (eb[j], 0)),
            pl.BlockSpec((1, H, D), lambda j, eb, ee, es, en, ef: (ee[j], 0, 0),
                         pipeline_mode=pl.Buffered(buffer_count=1)),
            pl.BlockSpec((1, H, D), lambda j, eb, ee, es, en, ef: (ee[j], 0, 0),
                         pipeline_mode=pl.Buffered(buffer_count=1)),
            pl.BlockSpec((1, D, H), lambda j, eb, ee, es, en, ef: (ee[j], 0, 0),
                         pipeline_mode=pl.Buffered(buffer_count=2)),
        ],
        out_specs=pl.BlockSpec((M, D), lambda j, eb, ee, es, en, ef: (eb[j], 0)),
    )
    return pl.pallas_call(
        body,
        grid_spec=grid_spec,
        out_shape=jax.ShapeDtypeStruct((TS, D), jnp.float32),
        compiler_params=pltpu.CompilerParams(
            dimension_semantics=("arbitrary",),
        ),
    )(eb, ee, es, en, ef, Xs, W1, W3, W2)


def _weighted_add(G3, tw):
    T, _, D = G3.shape
    BK = min(512, T)

    def body(g_ref, w_ref, o_ref):
        g = g_ref[...]
        w = w_ref[...]
        o_ref[...] = g[:, 0, :] * w[:, 0:1] + g[:, 1, :] * w[:, 1:2]

    return pl.pallas_call(
        body,
        grid=(T // BK,),
        in_specs=[
            pl.BlockSpec((BK, TOPK, D), lambda t: (t, 0, 0)),
            pl.BlockSpec((BK, TOPK), lambda t: (t, 0)),
        ],
        out_specs=pl.BlockSpec((BK, D), lambda t: (t, 0)),
        out_shape=jax.ShapeDtypeStruct((T, D), jnp.float32),
    )(G3, tw)


def _entry_tables(off_i, M, NB, NE, E):
    cnt = off_i[1:] - off_i[:-1]
    fb = jnp.where(cnt > 0, off_i[:-1] // M, 0)
    lb = jnp.where(cnt > 0, (off_i[1:] - 1) // M, -1)
    ne = jnp.maximum(lb - fb + 1, 0)
    basec = jnp.concatenate([jnp.zeros((1,), jnp.int32), jnp.cumsum(ne)])
    total = basec[E]
    j = jnp.arange(NE, dtype=jnp.int32)
    e_j = jnp.sum((j[:, None] >= basec[None, 1:]).astype(jnp.int32), axis=1)
    valid = j < total
    e_c = jnp.minimum(e_j, E - 1)
    blk = fb[e_c] + (j - basec[e_c])
    blk = jnp.where(valid, blk, NB - 1)
    es = jnp.where(valid, off_i[e_c], 0)
    en = jnp.where(valid, off_i[e_c + 1], 0)
    ee = jnp.where(valid, e_c, 0)
    prev = jnp.concatenate([jnp.full((1,), -1, jnp.int32), blk[:-1]])
    ef = ((blk != prev) & valid).astype(jnp.int32)
    return (blk.astype(jnp.int32), ee.astype(jnp.int32), es.astype(jnp.int32),
            en.astype(jnp.int32), ef)


def kernel(x, gate_w, W1, W2, W3):
    B, S, D = x.shape
    E = gate_w.shape[0]
    T = B * S
    TS = T * TOPK
    M = min(256, TS)
    NB = TS // M
    NE = NB + E - 1

    xf = x.reshape(T, D)
    logits = xf @ gate_w.T
    tw, ti, counts = _gating(logits)

    counts_i = counts[0].astype(jnp.int32)
    off_i = jnp.concatenate(
        [jnp.zeros((1,), jnp.int32), jnp.cumsum(counts_i)])
    off_f = off_i[:-1].astype(jnp.float32).reshape(1, E)

    flat_e = ti.reshape(TS, 1)
    dest = _routing(flat_e, off_f, E)
    dest_flat = dest.reshape(TS)

    d2 = dest.reshape(T, TOPK)
    Xs = _dispatch_sc(xf, d2[:, 0], d2[:, 1], TS)

    tabs = _entry_tables(off_i, M, NB, NE, E)
    Ys = _ffn(Xs, W1, W2, W3, tabs, M, NB, NE)

    G = _gather_sc(Ys, dest_flat)
    y = _weighted_add(G.reshape(T, TOPK, D), tw)
    return y.reshape(B, S, D)
